# Initial kernel scaffold; baseline (speedup 1.0000x reference)
import jax
import jax.numpy as jnp
from jax.experimental import pallas as pl


def kernel(x, mean_list, scale_list, weight_logits):
    raise NotImplementedError("write your pallas kernel here")


# speedup vs baseline: 2.7259x; 2.7259x over previous
import math

import jax
import jax.numpy as jnp
from jax.experimental import pallas as pl

_TB = 256


def _mog_logprob_kernel(x_ref, mean_ref, scale_ref, wl_ref, out_ref):
    x = x_ref[...]
    wl = wl_ref[...]
    log_w = wl - jax.nn.logsumexp(wl, axis=-1, keepdims=True)

    k_tot = mean_ref.shape[0]
    half_log_2pi = 0.5 * math.log(2.0 * math.pi)

    m = None
    s = None
    for k in range(k_tot):
        sc = jax.nn.softplus(scale_ref[k, :])[None, :]
        mu = mean_ref[k, :][None, :]
        d = (x - mu) / sc
        v = -0.5 * d * d - jnp.log(sc) - half_log_2pi + log_w[0:1, k : k + 1]
        if m is None:
            m = v
            s = jnp.ones_like(v)
        else:
            m_new = jnp.maximum(m, v)
            s = s * jnp.exp(m - m_new) + jnp.exp(v - m_new)
            m = m_new
    out_ref[...] = m + jnp.log(s)


def kernel(x, mean_list, scale_list, weight_logits):
    b, z = x.shape
    k = mean_list.shape[-1]
    mean_t = mean_list[0].T
    scale_t = scale_list[0].T

    grid = (b // _TB,)
    return pl.pallas_call(
        _mog_logprob_kernel,
        grid=grid,
        in_specs=[
            pl.BlockSpec((_TB, z), lambda i: (i, 0)),
            pl.BlockSpec((k, z), lambda i: (0, 0)),
            pl.BlockSpec((k, z), lambda i: (0, 0)),
            pl.BlockSpec((1, k), lambda i: (0, 0)),
        ],
        out_specs=pl.BlockSpec((_TB, z), lambda i: (i, 0)),
        out_shape=jax.ShapeDtypeStruct((b, z), x.dtype),
    )(x, mean_t, scale_t, weight_logits)


# speedup vs baseline: 4.4572x; 1.6351x over previous
import math

import jax
import jax.numpy as jnp
from jax.experimental import pallas as pl

_TB = 256


def _mog_logprob_kernel(x_ref, mean_ref, scale_ref, wl_ref, out_ref):
    x = x_ref[...]
    wl = wl_ref[...]
    log_w = wl - jax.nn.logsumexp(wl, axis=-1, keepdims=True)

    k_tot = mean_ref.shape[0]
    half_log_2pi = 0.5 * math.log(2.0 * math.pi)

    x2 = x * x
    vs = []
    for k in range(k_tot):
        sc = jax.nn.softplus(scale_ref[k, :])[None, :]
        mu = mean_ref[k, :][None, :]
        q = -0.5 / (sc * sc)
        b = -2.0 * q * mu
        a = q * mu * mu - jnp.log(sc) - half_log_2pi + log_w[0:1, k : k + 1]
        vs.append(a + b * x + q * x2)
    m = vs[0]
    for k in range(1, k_tot):
        m = jnp.maximum(m, vs[k])
    s = jnp.exp(vs[0] - m)
    for k in range(1, k_tot):
        s = s + jnp.exp(vs[k] - m)
    out_ref[...] = m + jnp.log(s)


def kernel(x, mean_list, scale_list, weight_logits):
    b, z = x.shape
    k = mean_list.shape[-1]
    mean_t = mean_list[0].T
    scale_t = scale_list[0].T

    grid = (b // _TB,)
    return pl.pallas_call(
        _mog_logprob_kernel,
        grid=grid,
        in_specs=[
            pl.BlockSpec((_TB, z), lambda i: (i, 0)),
            pl.BlockSpec((k, z), lambda i: (0, 0)),
            pl.BlockSpec((k, z), lambda i: (0, 0)),
            pl.BlockSpec((1, k), lambda i: (0, 0)),
        ],
        out_specs=pl.BlockSpec((_TB, z), lambda i: (i, 0)),
        out_shape=jax.ShapeDtypeStruct((b, z), x.dtype),
    )(x, mean_t, scale_t, weight_logits)


# speedup vs baseline: 4.4721x; 1.0034x over previous
import math

import jax
import jax.numpy as jnp
from jax.experimental import pallas as pl

_TB = 256


def _mog_logprob_kernel(x_ref, mean_ref, scale_ref, wl_ref, out_ref):
    x = x_ref[...]
    wl = wl_ref[...]
    log_w = wl - jax.nn.logsumexp(wl, axis=-1, keepdims=True)

    k_tot = mean_ref.shape[0]
    half_log_2pi = 0.5 * math.log(2.0 * math.pi)

    x2 = x * x
    coef = []
    for k in range(k_tot):
        sc = jax.nn.softplus(scale_ref[k, :])[None, :]
        mu = mean_ref[k, :][None, :]
        q = -0.5 / (sc * sc)
        b = -2.0 * q * mu
        a = q * mu * mu - jnp.log(sc) - half_log_2pi + log_w[0:1, k : k + 1]
        coef.append((a, b, q))

    m = None
    for a, b, q in coef:
        v = a + b * x + q * x2
        m = v if m is None else jnp.maximum(m, v)
    s = None
    for a, b, q in coef:
        e = jnp.exp(a + b * x + q * x2 - m)
        s = e if s is None else s + e
    out_ref[...] = m + jnp.log(s)


def kernel(x, mean_list, scale_list, weight_logits):
    b, z = x.shape
    k = mean_list.shape[-1]
    mean_t = mean_list[0].T
    scale_t = scale_list[0].T

    grid = (b // _TB,)
    return pl.pallas_call(
        _mog_logprob_kernel,
        grid=grid,
        in_specs=[
            pl.BlockSpec((_TB, z), lambda i: (i, 0)),
            pl.BlockSpec((k, z), lambda i: (0, 0)),
            pl.BlockSpec((k, z), lambda i: (0, 0)),
            pl.BlockSpec((1, k), lambda i: (0, 0)),
        ],
        out_specs=pl.BlockSpec((_TB, z), lambda i: (i, 0)),
        out_shape=jax.ShapeDtypeStruct((b, z), x.dtype),
    )(x, mean_t, scale_t, weight_logits)
